# Initial kernel scaffold; baseline (speedup 1.0000x reference)
import jax
import jax.numpy as jnp
from jax.experimental import pallas as pl


def kernel(x, ht, r_tensor, queries, W_ent, b_ent, rel_emb, rel_transe, Wf0, bf0, Wb0, bb0, lnng0, lnnb0, We0, be0, lneg0, lneb0, Wf1, bf1, Wb1, bb1, lnng1, lnnb1, We1, be1, lneg1, lneb1):
    raise NotImplementedError("write your pallas kernel here")


# speedup vs baseline: 1.4960x; 1.4960x over previous
import jax
import jax.numpy as jnp
from jax import lax
from jax.experimental import pallas as pl
from jax.experimental.pallas import tpu as pltpu
from jax.experimental.pallas import tpu_sc as plsc

D = 128
N = 10000
N_PAD = 10240
TRASH = 10000
E = 160000
E_PAD = 163840
REL_PAD = 512
NC = 2
NS = 16
NW = NC * NS
CHUNK = 128
CPT = E_PAD // (NW * CHUNK)
AGG_CHUNK = 64
AGG_CPT = E_PAD // (NW * AGG_CHUNK)
NB = 512

_F32 = jnp.float32


def _lrelu(v):
    return jnp.maximum(v, 0.0) + 0.01 * jnp.minimum(v, 0.0)


def _lnorm(v, g, b, eps=1e-5):
    mu = jnp.mean(v, axis=-1, keepdims=True)
    var = jnp.mean((v - mu) ** 2, axis=-1, keepdims=True)
    return (v - mu) / jnp.sqrt(var + eps) * g + b


def _full(shape):
    return pl.BlockSpec(shape, lambda i: tuple(0 for _ in shape))


def _tc_prep(xp, w_ent, b_ent2, wf0h, wb0h):
    def body(x_ref, we_ref, be_ref, wf_ref, wb_ref, h_ref, tf_ref, tb_ref):
        h = _lrelu(jnp.dot(x_ref[...], we_ref[...],
                           preferred_element_type=_F32) + be_ref[...])
        h_ref[...] = h
        tf_ref[...] = jnp.dot(h, wf_ref[...], preferred_element_type=_F32)
        tb_ref[...] = jnp.dot(h, wb_ref[...], preferred_element_type=_F32)

    out = jax.ShapeDtypeStruct((N_PAD, D), _F32)
    return pl.pallas_call(
        body,
        grid=(N_PAD // NB,),
        in_specs=[pl.BlockSpec((NB, D), lambda i: (i, 0)),
                  _full((D, D)), _full((1, D)), _full((D, D)), _full((D, D))],
        out_specs=[pl.BlockSpec((NB, D), lambda i: (i, 0))] * 3,
        out_shape=[out, out, out],
    )(xp, w_ent, b_ent2, wf0h, wb0h)


def _tc_rel(relp, wf0e, bf02, wb0e, bb02, we0m, be02):
    def body(r_ref, wf_ref, bf_ref, wb_ref, bb_ref, we_ref, be_ref,
             rf_ref, rb_ref, re_ref):
        r = r_ref[...]
        rf_ref[...] = jnp.dot(r, wf_ref[...], preferred_element_type=_F32) + bf_ref[...]
        rb_ref[...] = jnp.dot(r, wb_ref[...], preferred_element_type=_F32) + bb_ref[...]
        re_ref[...] = jnp.dot(r, we_ref[...], preferred_element_type=_F32) + be_ref[...]

    out = jax.ShapeDtypeStruct((REL_PAD, D), _F32)
    return pl.pallas_call(
        body,
        grid=(1,),
        in_specs=[_full((REL_PAD, D)), _full((D, D)), _full((1, D)),
                  _full((D, D)), _full((1, D)), _full((D, D)), _full((1, D))],
        out_specs=[_full((REL_PAD, D))] * 3,
        out_shape=[out, out, out],
    )(relp, wf0e, bf02, wb0e, bb02, we0m, be02)


def _tc_update0(aggp, cntp, h0, g2, b2, we0h, we0t, wf1h, bf12, wb1h, bb12):
    def body(a_ref, c_ref, h_ref, g_ref, b_ref, weh_ref, wet_ref,
             wf_ref, bf_ref, wb_ref, bb_ref,
             h1_ref, teh_ref, tet_ref, tf_ref, tb_ref):
        agg = a_ref[0] + a_ref[1]
        cnt = (c_ref[0] + c_ref[1]).reshape(NB, 1)
        h1 = _lnorm(_lrelu(agg / jnp.maximum(cnt, 1.0)) + h_ref[...],
                    g_ref[...], b_ref[...])
        h1_ref[...] = h1
        teh_ref[...] = jnp.dot(h1, weh_ref[...], preferred_element_type=_F32)
        tet_ref[...] = jnp.dot(h1, wet_ref[...], preferred_element_type=_F32)
        tf_ref[...] = jnp.dot(h1, wf_ref[...], preferred_element_type=_F32) + bf_ref[...]
        tb_ref[...] = jnp.dot(h1, wb_ref[...], preferred_element_type=_F32) + bb_ref[...]

    out = jax.ShapeDtypeStruct((N_PAD, D), _F32)
    return pl.pallas_call(
        body,
        grid=(N_PAD // NB,),
        in_specs=[pl.BlockSpec((NC, NB, D), lambda i: (0, i, 0)),
                  pl.BlockSpec((NC, NB), lambda i: (0, i)),
                  pl.BlockSpec((NB, D), lambda i: (i, 0)),
                  _full((1, D)), _full((1, D)), _full((D, D)), _full((D, D)),
                  _full((D, D)), _full((1, D)), _full((D, D)), _full((1, D))],
        out_specs=[pl.BlockSpec((NB, D), lambda i: (i, 0))] * 5,
        out_shape=[out] * 5,
    )(aggp, cntp, h0, g2, b2, we0h, we0t, wf1h, bf12, wb1h, bb12)


def _tc_update1(aggp, cntp, h1, g2, b2):
    def body(a_ref, c_ref, h_ref, g_ref, b_ref, h2_ref):
        agg = a_ref[0] + a_ref[1]
        cnt = (c_ref[0] + c_ref[1]).reshape(NB, 1)
        h2_ref[...] = _lnorm(_lrelu(agg / jnp.maximum(cnt, 1.0)) + h_ref[...],
                             g_ref[...], b_ref[...])

    return pl.pallas_call(
        body,
        grid=(N_PAD // NB,),
        in_specs=[pl.BlockSpec((NC, NB, D), lambda i: (0, i, 0)),
                  pl.BlockSpec((NC, NB), lambda i: (0, i)),
                  pl.BlockSpec((NB, D), lambda i: (i, 0)),
                  _full((1, D)), _full((1, D))],
        out_specs=pl.BlockSpec((NB, D), lambda i: (i, 0)),
        out_shape=jax.ShapeDtypeStruct((N_PAD, D), _F32),
    )(aggp, cntp, h1, g2, b2)


def _tc_msg1(qe, g2, b2, wf1e, wb1e):
    def body(q_ref, g_ref, b_ref, wf_ref, wb_ref, mf_ref, mb_ref):
        e1 = _lnorm(q_ref[...], g_ref[...], b_ref[...])
        mf_ref[...] = jnp.dot(e1, wf_ref[...], preferred_element_type=_F32)
        mb_ref[...] = jnp.dot(e1, wb_ref[...], preferred_element_type=_F32)

    out = jax.ShapeDtypeStruct((E_PAD, D), _F32)
    return pl.pallas_call(
        body,
        grid=(E_PAD // NB,),
        in_specs=[pl.BlockSpec((NB, D), lambda i: (i, 0)),
                  _full((1, D)), _full((1, D)), _full((D, D)), _full((D, D))],
        out_specs=[pl.BlockSpec((NB, D), lambda i: (i, 0))] * 2,
        out_shape=[out, out],
    )(qe, g2, b2, wf1e, wb1e)


def _tc_final(d2p, qf):
    def body(d_ref, q_ref, o_ref):
        s = jnp.sum(d_ref[...], axis=1)
        o_ref[...] = -jnp.sqrt(s + 1e-12) * q_ref[...]

    return pl.pallas_call(
        body,
        grid=(E_PAD // NB,),
        in_specs=[pl.BlockSpec((NB, 16), lambda i: (i, 0)),
                  pl.BlockSpec((NB,), lambda i: (i,))],
        out_specs=pl.BlockSpec((NB,), lambda i: (i,)),
        out_shape=jax.ShapeDtypeStruct((E_PAD,), _F32),
    )(d2p, qf)


def _sc_mesh():
    return plsc.VectorSubcoreMesh(core_axis_name="c", subcore_axis_name="s")


def _sc_agg(tf, tb, sf, sb, i0, i1, ir, zrows, zcnt, orows, layer0):
    outs = [jax.ShapeDtypeStruct((NC * N_PAD, D), _F32)]
    scratch = [
        pltpu.VMEM((CHUNK, D), _F32),
        pltpu.VMEM((CHUNK,), _F32),
        pltpu.VMEM((CHUNK,), jnp.int32),
        pltpu.VMEM((CHUNK,), jnp.int32),
        pltpu.VMEM((CHUNK,), jnp.int32),
        pltpu.VMEM_SHARED((N_PAD, D), _F32),
        pltpu.VMEM_SHARED((N_PAD,), _F32),
        pltpu.SemaphoreType.DMA,
    ]
    if layer0:
        outs.append(jax.ShapeDtypeStruct((NC * N_PAD,), _F32))

    def body(*refs):
        if layer0:
            (tf_h, tb_h, sf_h, sb_h, i0_h, i1_h, ir_h, z_h, zc_h, o_h,
             agg_o, cnt_o,
             buf, ones, ib0, ib1, ibr, agg_sh, cnt_sh, sem) = refs
        else:
            (tf_h, tb_h, sf_h, sb_h, i0_h, i1_h, z_h, agg_o,
             buf, ones, ib0, ib1, ibr, agg_sh, cnt_sh, sem) = refs
            ir_h = zc_h = o_h = cnt_o = None
        cid = lax.axis_index("c")
        sid = lax.axis_index("s")
        wid = sid * NC + cid
        rows = N_PAD // NS

        pltpu.sync_copy(z_h.at[pl.ds(0, rows)], agg_sh.at[pl.ds(sid * rows, rows)])
        if layer0:
            pltpu.sync_copy(zc_h.at[pl.ds(0, rows)],
                            cnt_sh.at[pl.ds(sid * rows, rows)])
            pltpu.sync_copy(o_h.at[pl.ds(0, CHUNK)], ones)

        plsc.subcore_barrier()

        base0 = wid * CPT * CHUNK

        @pl.loop(0, CPT)
        def _(k):
            base = base0 + k * CHUNK
            pltpu.sync_copy(i0_h.at[pl.ds(base, CHUNK)], ib0)
            pltpu.sync_copy(i1_h.at[pl.ds(base, CHUNK)], ib1)
            if layer0:
                pltpu.sync_copy(ir_h.at[pl.ds(base, CHUNK)], ibr)
            pltpu.async_copy(tf_h.at[ib0], buf, sem).wait()
            pltpu.sync_copy(buf, agg_sh.at[ib1], add=True)
            if layer0:
                pltpu.async_copy(sf_h.at[ibr], buf, sem).wait()
            else:
                pltpu.async_copy(sf_h.at[pl.ds(base, CHUNK)], buf, sem).wait()
            pltpu.sync_copy(buf, agg_sh.at[ib1], add=True)
            pltpu.async_copy(tb_h.at[ib1], buf, sem).wait()
            pltpu.sync_copy(buf, agg_sh.at[ib0], add=True)
            if layer0:
                pltpu.async_copy(sb_h.at[ibr], buf, sem).wait()
            else:
                pltpu.async_copy(sb_h.at[pl.ds(base, CHUNK)], buf, sem).wait()
            pltpu.sync_copy(buf, agg_sh.at[ib0], add=True)
            if layer0:
                pltpu.sync_copy(ones, cnt_sh.at[ib1], add=True)
                pltpu.sync_copy(ones, cnt_sh.at[ib0], add=True)

        plsc.subcore_barrier()

        @pl.loop(0, rows // CHUNK)
        def _(j):
            lo = sid * rows + j * CHUNK
            pltpu.sync_copy(agg_sh.at[pl.ds(lo, CHUNK)],
                            agg_o.at[pl.ds(cid * N_PAD + lo, CHUNK)])
        if layer0:
            pltpu.sync_copy(cnt_sh.at[pl.ds(sid * rows, rows)],
                            cnt_o.at[pl.ds(cid * N_PAD + sid * rows, rows)])

    k = pl.kernel(body, out_type=tuple(outs) if layer0 else outs[0],
                  mesh=_sc_mesh(), scratch_types=scratch)
    if layer0:
        a, c = k(tf, tb, sf, sb, i0, i1, ir, zrows, zcnt, orows)
        return a.reshape(NC, N_PAD, D), c.reshape(NC, N_PAD)
    return k(tf, tb, sf, sb, i0, i1, zrows).reshape(NC, N_PAD, D)


def _sc_edgeq(teh, tet, re0, relp, i0, i1, ir):
    def body(teh_h, tet_h, re_h, rel_h, i0_h, i1_h, ir_h, q_o,
             bufA, bufB, bufC, bufD, ib0, ib1, ibr, semA, semB, semC, semD):
        cid = lax.axis_index("c")
        sid = lax.axis_index("s")
        wid = sid * NC + cid
        base0 = wid * CPT * CHUNK

        @pl.loop(0, CPT)
        def _(k):
            base = base0 + k * CHUNK
            pltpu.sync_copy(i0_h.at[pl.ds(base, CHUNK)], ib0)
            pltpu.sync_copy(i1_h.at[pl.ds(base, CHUNK)], ib1)
            pltpu.sync_copy(ir_h.at[pl.ds(base, CHUNK)], ibr)
            cA = pltpu.async_copy(teh_h.at[ib0], bufA, semA)
            cB = pltpu.async_copy(tet_h.at[ib1], bufB, semB)
            cC = pltpu.async_copy(re_h.at[ibr], bufC, semC)
            cD = pltpu.async_copy(rel_h.at[ibr], bufD, semD)
            cA.wait()
            cB.wait()
            cC.wait()
            cD.wait()

            @pl.loop(0, CHUNK)
            def _(e):
                for c in range(D // 16):
                    sl = pl.ds(c * 16, 16)
                    g = bufA[e, sl] + bufB[e, sl] + bufC[e, sl]
                    bufA[e, sl] = (jnp.maximum(g, 0.0)
                                   + 0.01 * jnp.minimum(g, 0.0)) + bufD[e, sl]

            pltpu.sync_copy(bufA, q_o.at[pl.ds(base, CHUNK)])

    scratch = ([pltpu.VMEM((CHUNK, D), _F32)] * 4
               + [pltpu.VMEM((CHUNK,), jnp.int32)] * 3
               + [pltpu.SemaphoreType.DMA] * 4)
    k = pl.kernel(body, out_type=jax.ShapeDtypeStruct((E_PAD, D), _F32),
                  mesh=_sc_mesh(), scratch_types=scratch)
    return k(teh, tet, re0, relp, i0, i1, ir)


def _sc_score(h2, rtp, i0, i1, ir):
    def body(h_h, rt_h, i0_h, i1_h, ir_h, d2_o,
             bufA, bufB, bufC, sbuf, ib0, ib1, ibr, semA, semB, semC):
        cid = lax.axis_index("c")
        sid = lax.axis_index("s")
        wid = sid * NC + cid
        base0 = wid * CPT * CHUNK

        @pl.loop(0, CPT)
        def _(k):
            base = base0 + k * CHUNK
            pltpu.sync_copy(i0_h.at[pl.ds(base, CHUNK)], ib0)
            pltpu.sync_copy(i1_h.at[pl.ds(base, CHUNK)], ib1)
            pltpu.sync_copy(ir_h.at[pl.ds(base, CHUNK)], ibr)
            cA = pltpu.async_copy(h_h.at[ib0], bufA, semA)
            cB = pltpu.async_copy(h_h.at[ib1], bufB, semB)
            cC = pltpu.async_copy(rt_h.at[ibr], bufC, semC)
            cA.wait()
            cB.wait()
            cC.wait()

            @pl.loop(0, CHUNK)
            def _(e):
                acc = jnp.zeros((16,), _F32)
                for c in range(D // 16):
                    sl = pl.ds(c * 16, 16)
                    dd = bufA[e, sl] + bufC[e, sl] - bufB[e, sl]
                    acc = acc + dd * dd
                sbuf[pl.ds(e * 16, 16)] = acc

            pltpu.sync_copy(sbuf, d2_o.at[pl.ds(base * 16, CHUNK * 16)])

    scratch = ([pltpu.VMEM((CHUNK, D), _F32)] * 3
               + [pltpu.VMEM((CHUNK * 16,), _F32)]
               + [pltpu.VMEM((CHUNK,), jnp.int32)] * 3
               + [pltpu.SemaphoreType.DMA] * 3)
    k = pl.kernel(body, out_type=jax.ShapeDtypeStruct((E_PAD * 16,), _F32),
                  mesh=_sc_mesh(), scratch_types=scratch)
    return k(h2, rtp, i0, i1, ir).reshape(E_PAD, 16)


def kernel(x, ht, r_tensor, queries, W_ent, b_ent, rel_emb, rel_transe,
           Wf0, bf0, Wb0, bb0, lnng0, lnnb0, We0, be0, lneg0, lneb0,
           Wf1, bf1, Wb1, bb1, lnng1, lnnb1, We1, be1, lneg1, lneb1):
    pad_e = E_PAD - E
    xp = jnp.pad(x, ((0, N_PAD - N), (0, 0)))
    i0 = jnp.concatenate([ht[:, 0], jnp.full((pad_e,), TRASH, jnp.int32)])
    i1 = jnp.concatenate([ht[:, 1], jnp.full((pad_e,), TRASH, jnp.int32)])
    ir = jnp.concatenate([r_tensor, jnp.zeros((pad_e,), jnp.int32)])
    relp = jnp.pad(rel_emb, ((0, REL_PAD - rel_emb.shape[0]), (0, 0)))
    rtp = jnp.pad(rel_transe, ((0, REL_PAD - rel_transe.shape[0]), (0, 0)))
    qf = jnp.pad(queries.astype(_F32), (0, pad_e))

    row = lambda v: v.reshape(1, D)

    h0, tf0, tb0 = _tc_prep(xp, W_ent, row(b_ent), Wf0[:D], Wb0[:D])
    rf0, rb0, re0 = _tc_rel(relp, Wf0[D:], row(bf0), Wb0[D:], row(bb0),
                            We0[D:2 * D], row(be0))

    zrows = jnp.zeros((N_PAD // NS, D), _F32)
    zcnt = jnp.zeros((N_PAD // NS,), _F32)
    orows = jnp.ones((CHUNK,), _F32)

    aggp, cntp = _sc_agg(tf0, tb0, rf0, rb0, i0, i1, ir,
                         zrows, zcnt, orows, layer0=True)

    h1, teh, tet, tf1, tb1 = _tc_update0(
        aggp, cntp, h0, row(lnng0), row(lnnb0),
        We0[:D], We0[2 * D:], Wf1[:D], row(bf1), Wb1[:D], row(bb1))

    qe = _sc_edgeq(teh, tet, re0, relp, i0, i1, ir)
    mf, mb = _tc_msg1(qe, row(lneg0), row(lneb0), Wf1[D:], Wb1[D:])

    aggp1 = _sc_agg(tf1, tb1, mf, mb, i0, i1, None,
                    zrows, None, None, layer0=False)

    h2 = _tc_update1(aggp1, cntp, h1, row(lnng1), row(lnnb1))

    d2p = _sc_score(h2, rtp, i0, i1, ir)
    out = _tc_final(d2p, qf)
    return out[:E]


# speedup vs baseline: 1.7115x; 1.1440x over previous
import jax
import jax.numpy as jnp
from jax import lax
from jax.experimental import pallas as pl
from jax.experimental.pallas import tpu as pltpu
from jax.experimental.pallas import tpu_sc as plsc

D = 128
N = 10000
N_PAD = 10240
TRASH = 10000
E = 160000
E_PAD = 163840
REL_PAD = 512
NC = 2
NS = 16
NW = NC * NS
CHUNK = 128
CPT = E_PAD // (NW * CHUNK)
AGG_CHUNK = 64
AGG_CPT = E_PAD // (NW * AGG_CHUNK)
NB = 512

_F32 = jnp.float32


def _lrelu(v):
    return jnp.maximum(v, 0.0) + 0.01 * jnp.minimum(v, 0.0)


def _lnorm(v, g, b, eps=1e-5):
    mu = jnp.mean(v, axis=-1, keepdims=True)
    var = jnp.mean((v - mu) ** 2, axis=-1, keepdims=True)
    return (v - mu) / jnp.sqrt(var + eps) * g + b


def _full(shape):
    return pl.BlockSpec(shape, lambda i: tuple(0 for _ in shape))


def _tc_prep(xp, w_ent, b_ent2, wf0h, wb0h):
    def body(x_ref, we_ref, be_ref, wf_ref, wb_ref, h_ref, tf_ref, tb_ref):
        h = _lrelu(jnp.dot(x_ref[...], we_ref[...],
                           preferred_element_type=_F32) + be_ref[...])
        h_ref[...] = h
        tf_ref[...] = jnp.dot(h, wf_ref[...], preferred_element_type=_F32)
        tb_ref[...] = jnp.dot(h, wb_ref[...], preferred_element_type=_F32)

    out = jax.ShapeDtypeStruct((N_PAD, D), _F32)
    return pl.pallas_call(
        body,
        grid=(N_PAD // NB,),
        in_specs=[pl.BlockSpec((NB, D), lambda i: (i, 0)),
                  _full((D, D)), _full((1, D)), _full((D, D)), _full((D, D))],
        out_specs=[pl.BlockSpec((NB, D), lambda i: (i, 0))] * 3,
        out_shape=[out, out, out],
    )(xp, w_ent, b_ent2, wf0h, wb0h)


def _tc_rel(relp, wf0e, bf02, wb0e, bb02, we0m, be02):
    def body(r_ref, wf_ref, bf_ref, wb_ref, bb_ref, we_ref, be_ref,
             rf_ref, rb_ref, re_ref):
        r = r_ref[...]
        rf_ref[...] = jnp.dot(r, wf_ref[...], preferred_element_type=_F32) + bf_ref[...]
        rb_ref[...] = jnp.dot(r, wb_ref[...], preferred_element_type=_F32) + bb_ref[...]
        re_ref[...] = jnp.dot(r, we_ref[...], preferred_element_type=_F32) + be_ref[...]

    out = jax.ShapeDtypeStruct((REL_PAD, D), _F32)
    return pl.pallas_call(
        body,
        grid=(1,),
        in_specs=[_full((REL_PAD, D)), _full((D, D)), _full((1, D)),
                  _full((D, D)), _full((1, D)), _full((D, D)), _full((1, D))],
        out_specs=[_full((REL_PAD, D))] * 3,
        out_shape=[out, out, out],
    )(relp, wf0e, bf02, wb0e, bb02, we0m, be02)


def _tc_update0(aggp, cntp, h0, g2, b2, we0h, we0t, wf1h, bf12, wb1h, bb12):
    def body(a_ref, c_ref, h_ref, g_ref, b_ref, weh_ref, wet_ref,
             wf_ref, bf_ref, wb_ref, bb_ref,
             h1_ref, teh_ref, tet_ref, tf_ref, tb_ref):
        agg = a_ref[0] + a_ref[1]
        cnt = (c_ref[0] + c_ref[1]).reshape(NB, 1)
        h1 = _lnorm(_lrelu(agg / jnp.maximum(cnt, 1.0)) + h_ref[...],
                    g_ref[...], b_ref[...])
        h1_ref[...] = h1
        teh_ref[...] = jnp.dot(h1, weh_ref[...], preferred_element_type=_F32)
        tet_ref[...] = jnp.dot(h1, wet_ref[...], preferred_element_type=_F32)
        tf_ref[...] = jnp.dot(h1, wf_ref[...], preferred_element_type=_F32) + bf_ref[...]
        tb_ref[...] = jnp.dot(h1, wb_ref[...], preferred_element_type=_F32) + bb_ref[...]

    out = jax.ShapeDtypeStruct((N_PAD, D), _F32)
    return pl.pallas_call(
        body,
        grid=(N_PAD // NB,),
        in_specs=[pl.BlockSpec((NC, NB, D), lambda i: (0, i, 0)),
                  pl.BlockSpec((NC, NB), lambda i: (0, i)),
                  pl.BlockSpec((NB, D), lambda i: (i, 0)),
                  _full((1, D)), _full((1, D)), _full((D, D)), _full((D, D)),
                  _full((D, D)), _full((1, D)), _full((D, D)), _full((1, D))],
        out_specs=[pl.BlockSpec((NB, D), lambda i: (i, 0))] * 5,
        out_shape=[out] * 5,
    )(aggp, cntp, h0, g2, b2, we0h, we0t, wf1h, bf12, wb1h, bb12)


def _tc_update1(aggp, cntp, h1, g2, b2):
    def body(a_ref, c_ref, h_ref, g_ref, b_ref, h2_ref):
        agg = a_ref[0] + a_ref[1]
        cnt = (c_ref[0] + c_ref[1]).reshape(NB, 1)
        h2_ref[...] = _lnorm(_lrelu(agg / jnp.maximum(cnt, 1.0)) + h_ref[...],
                             g_ref[...], b_ref[...])

    return pl.pallas_call(
        body,
        grid=(N_PAD // NB,),
        in_specs=[pl.BlockSpec((NC, NB, D), lambda i: (0, i, 0)),
                  pl.BlockSpec((NC, NB), lambda i: (0, i)),
                  pl.BlockSpec((NB, D), lambda i: (i, 0)),
                  _full((1, D)), _full((1, D))],
        out_specs=pl.BlockSpec((NB, D), lambda i: (i, 0)),
        out_shape=jax.ShapeDtypeStruct((N_PAD, D), _F32),
    )(aggp, cntp, h1, g2, b2)


def _tc_msg1(qe, g2, b2, wf1e, wb1e):
    def body(q_ref, g_ref, b_ref, wf_ref, wb_ref, mf_ref, mb_ref):
        e1 = _lnorm(q_ref[...], g_ref[...], b_ref[...])
        mf_ref[...] = jnp.dot(e1, wf_ref[...], preferred_element_type=_F32)
        mb_ref[...] = jnp.dot(e1, wb_ref[...], preferred_element_type=_F32)

    out = jax.ShapeDtypeStruct((E_PAD, D), _F32)
    return pl.pallas_call(
        body,
        grid=(E_PAD // NB,),
        in_specs=[pl.BlockSpec((NB, D), lambda i: (i, 0)),
                  _full((1, D)), _full((1, D)), _full((D, D)), _full((D, D))],
        out_specs=[pl.BlockSpec((NB, D), lambda i: (i, 0))] * 2,
        out_shape=[out, out],
    )(qe, g2, b2, wf1e, wb1e)


def _tc_final(d2p, qf):
    def body(d_ref, q_ref, o_ref):
        s = jnp.sum(d_ref[...], axis=1)
        o_ref[...] = -jnp.sqrt(s + 1e-12) * q_ref[...]

    return pl.pallas_call(
        body,
        grid=(E_PAD // NB,),
        in_specs=[pl.BlockSpec((NB, 16), lambda i: (i, 0)),
                  pl.BlockSpec((NB,), lambda i: (i,))],
        out_specs=pl.BlockSpec((NB,), lambda i: (i,)),
        out_shape=jax.ShapeDtypeStruct((E_PAD,), _F32),
    )(d2p, qf)


def _sc_mesh():
    return plsc.VectorSubcoreMesh(core_axis_name="c", subcore_axis_name="s")


def _sc_agg(tf, tb, sf, sb, i0, i1, ir, zrows, zcnt, orows, layer0):
    outs = [jax.ShapeDtypeStruct((NC * N_PAD, D), _F32)]
    scratch = [
        pltpu.VMEM((CHUNK, D), _F32),
        pltpu.VMEM((CHUNK, D), _F32),
        pltpu.VMEM((CHUNK,), _F32),
        pltpu.VMEM((CHUNK,), jnp.int32),
        pltpu.VMEM((CHUNK,), jnp.int32),
        pltpu.VMEM((CHUNK,), jnp.int32),
        pltpu.VMEM_SHARED((N_PAD, D), _F32),
        pltpu.VMEM_SHARED((N_PAD,), _F32),
        pltpu.SemaphoreType.DMA,
        pltpu.SemaphoreType.DMA,
    ]
    if layer0:
        outs.append(jax.ShapeDtypeStruct((NC * N_PAD,), _F32))

    def body(*refs):
        if layer0:
            (tf_h, tb_h, sf_h, sb_h, i0_h, i1_h, ir_h, z_h, zc_h, o_h,
             agg_o, cnt_o,
             bufA, bufB, ones, ib0, ib1, ibr, agg_sh, cnt_sh, semA, semB) = refs
        else:
            (tf_h, tb_h, sf_h, sb_h, i0_h, i1_h, z_h, agg_o,
             bufA, bufB, ones, ib0, ib1, ibr, agg_sh, cnt_sh, semA, semB) = refs
            ir_h = zc_h = o_h = cnt_o = None
        cid = lax.axis_index("c")
        sid = lax.axis_index("s")
        wid = sid * NC + cid
        rows = N_PAD // NS

        pltpu.sync_copy(z_h.at[pl.ds(0, rows)], agg_sh.at[pl.ds(sid * rows, rows)])
        if layer0:
            pltpu.sync_copy(zc_h.at[pl.ds(0, rows)],
                            cnt_sh.at[pl.ds(sid * rows, rows)])
            pltpu.sync_copy(o_h.at[pl.ds(0, CHUNK)], ones)

        plsc.subcore_barrier()

        base0 = wid * CPT * CHUNK

        @pl.loop(0, CPT)
        def _(k):
            base = base0 + k * CHUNK
            pltpu.sync_copy(i0_h.at[pl.ds(base, CHUNK)], ib0)
            pltpu.sync_copy(i1_h.at[pl.ds(base, CHUNK)], ib1)
            if layer0:
                pltpu.sync_copy(ir_h.at[pl.ds(base, CHUNK)], ibr)
            cA = pltpu.async_copy(tf_h.at[ib0], bufA, semA)
            if layer0:
                cB = pltpu.async_copy(sf_h.at[ibr], bufB, semB)
            else:
                cB = pltpu.async_copy(sf_h.at[pl.ds(base, CHUNK)], bufB, semB)
            cA.wait()
            pltpu.sync_copy(bufA, agg_sh.at[ib1], add=True)
            cB.wait()
            cA = pltpu.async_copy(tb_h.at[ib1], bufA, semA)
            pltpu.sync_copy(bufB, agg_sh.at[ib1], add=True)
            if layer0:
                cB = pltpu.async_copy(sb_h.at[ibr], bufB, semB)
            else:
                cB = pltpu.async_copy(sb_h.at[pl.ds(base, CHUNK)], bufB, semB)
            cA.wait()
            pltpu.sync_copy(bufA, agg_sh.at[ib0], add=True)
            cB.wait()
            pltpu.sync_copy(bufB, agg_sh.at[ib0], add=True)
            if layer0:
                pltpu.sync_copy(ones, cnt_sh.at[ib1], add=True)
                pltpu.sync_copy(ones, cnt_sh.at[ib0], add=True)

        plsc.subcore_barrier()

        @pl.loop(0, rows // CHUNK)
        def _(j):
            lo = sid * rows + j * CHUNK
            pltpu.sync_copy(agg_sh.at[pl.ds(lo, CHUNK)],
                            agg_o.at[pl.ds(cid * N_PAD + lo, CHUNK)])
        if layer0:
            pltpu.sync_copy(cnt_sh.at[pl.ds(sid * rows, rows)],
                            cnt_o.at[pl.ds(cid * N_PAD + sid * rows, rows)])

    k = pl.kernel(body, out_type=tuple(outs) if layer0 else outs[0],
                  mesh=_sc_mesh(), scratch_types=scratch)
    if layer0:
        a, c = k(tf, tb, sf, sb, i0, i1, ir, zrows, zcnt, orows)
        return a.reshape(NC, N_PAD, D), c.reshape(NC, N_PAD)
    return k(tf, tb, sf, sb, i0, i1, zrows).reshape(NC, N_PAD, D)


def _sc_edgeq(teh, tet, re0, relp, i0, i1, ir):
    def body(teh_h, tet_h, re_h, rel_h, i0_h, i1_h, ir_h, q_o,
             bufA, bufB, bufC, bufD, ib0, ib1, ibr, semA, semB, semC, semD):
        cid = lax.axis_index("c")
        sid = lax.axis_index("s")
        wid = sid * NC + cid
        base0 = wid * CPT * CHUNK

        @pl.loop(0, CPT)
        def _(k):
            base = base0 + k * CHUNK
            pltpu.sync_copy(i0_h.at[pl.ds(base, CHUNK)], ib0)
            pltpu.sync_copy(i1_h.at[pl.ds(base, CHUNK)], ib1)
            pltpu.sync_copy(ir_h.at[pl.ds(base, CHUNK)], ibr)
            cA = pltpu.async_copy(teh_h.at[ib0], bufA, semA)
            cB = pltpu.async_copy(tet_h.at[ib1], bufB, semB)
            cC = pltpu.async_copy(re_h.at[ibr], bufC, semC)
            cD = pltpu.async_copy(rel_h.at[ibr], bufD, semD)
            cA.wait()
            cB.wait()
            cC.wait()
            cD.wait()

            @pl.loop(0, CHUNK)
            def _(e):
                for c in range(D // 16):
                    sl = pl.ds(c * 16, 16)
                    g = bufA[e, sl] + bufB[e, sl] + bufC[e, sl]
                    bufA[e, sl] = (jnp.maximum(g, 0.0)
                                   + 0.01 * jnp.minimum(g, 0.0)) + bufD[e, sl]

            pltpu.sync_copy(bufA, q_o.at[pl.ds(base, CHUNK)])

    scratch = ([pltpu.VMEM((CHUNK, D), _F32)] * 4
               + [pltpu.VMEM((CHUNK,), jnp.int32)] * 3
               + [pltpu.SemaphoreType.DMA] * 4)
    k = pl.kernel(body, out_type=jax.ShapeDtypeStruct((E_PAD, D), _F32),
                  mesh=_sc_mesh(), scratch_types=scratch)
    return k(teh, tet, re0, relp, i0, i1, ir)


def _sc_score(h2, rtp, i0, i1, ir):
    def body(h_h, rt_h, i0_h, i1_h, ir_h, d2_o,
             bufA, bufB, bufC, sbuf, ib0, ib1, ibr, semA, semB, semC):
        cid = lax.axis_index("c")
        sid = lax.axis_index("s")
        wid = sid * NC + cid
        base0 = wid * CPT * CHUNK

        @pl.loop(0, CPT)
        def _(k):
            base = base0 + k * CHUNK
            pltpu.sync_copy(i0_h.at[pl.ds(base, CHUNK)], ib0)
            pltpu.sync_copy(i1_h.at[pl.ds(base, CHUNK)], ib1)
            pltpu.sync_copy(ir_h.at[pl.ds(base, CHUNK)], ibr)
            cA = pltpu.async_copy(h_h.at[ib0], bufA, semA)
            cB = pltpu.async_copy(h_h.at[ib1], bufB, semB)
            cC = pltpu.async_copy(rt_h.at[ibr], bufC, semC)
            cA.wait()
            cB.wait()
            cC.wait()

            @pl.loop(0, CHUNK)
            def _(e):
                acc = jnp.zeros((16,), _F32)
                for c in range(D // 16):
                    sl = pl.ds(c * 16, 16)
                    dd = bufA[e, sl] + bufC[e, sl] - bufB[e, sl]
                    acc = acc + dd * dd
                sbuf[pl.ds(e * 16, 16)] = acc

            pltpu.sync_copy(sbuf, d2_o.at[pl.ds(base * 16, CHUNK * 16)])

    scratch = ([pltpu.VMEM((CHUNK, D), _F32)] * 3
               + [pltpu.VMEM((CHUNK * 16,), _F32)]
               + [pltpu.VMEM((CHUNK,), jnp.int32)] * 3
               + [pltpu.SemaphoreType.DMA] * 3)
    k = pl.kernel(body, out_type=jax.ShapeDtypeStruct((E_PAD * 16,), _F32),
                  mesh=_sc_mesh(), scratch_types=scratch)
    return k(h2, rtp, i0, i1, ir).reshape(E_PAD, 16)


def kernel(x, ht, r_tensor, queries, W_ent, b_ent, rel_emb, rel_transe,
           Wf0, bf0, Wb0, bb0, lnng0, lnnb0, We0, be0, lneg0, lneb0,
           Wf1, bf1, Wb1, bb1, lnng1, lnnb1, We1, be1, lneg1, lneb1):
    pad_e = E_PAD - E
    xp = jnp.pad(x, ((0, N_PAD - N), (0, 0)))
    i0 = jnp.concatenate([ht[:, 0], jnp.full((pad_e,), TRASH, jnp.int32)])
    i1 = jnp.concatenate([ht[:, 1], jnp.full((pad_e,), TRASH, jnp.int32)])
    ir = jnp.concatenate([r_tensor, jnp.zeros((pad_e,), jnp.int32)])
    relp = jnp.pad(rel_emb, ((0, REL_PAD - rel_emb.shape[0]), (0, 0)))
    rtp = jnp.pad(rel_transe, ((0, REL_PAD - rel_transe.shape[0]), (0, 0)))
    qf = jnp.pad(queries.astype(_F32), (0, pad_e))

    row = lambda v: v.reshape(1, D)

    h0, tf0, tb0 = _tc_prep(xp, W_ent, row(b_ent), Wf0[:D], Wb0[:D])
    rf0, rb0, re0 = _tc_rel(relp, Wf0[D:], row(bf0), Wb0[D:], row(bb0),
                            We0[D:2 * D], row(be0))

    zrows = jnp.zeros((N_PAD // NS, D), _F32)
    zcnt = jnp.zeros((N_PAD // NS,), _F32)
    orows = jnp.ones((CHUNK,), _F32)

    aggp, cntp = _sc_agg(tf0, tb0, rf0, rb0, i0, i1, ir,
                         zrows, zcnt, orows, layer0=True)

    h1, teh, tet, tf1, tb1 = _tc_update0(
        aggp, cntp, h0, row(lnng0), row(lnnb0),
        We0[:D], We0[2 * D:], Wf1[:D], row(bf1), Wb1[:D], row(bb1))

    qe = _sc_edgeq(teh, tet, re0, relp, i0, i1, ir)
    mf, mb = _tc_msg1(qe, row(lneg0), row(lneb0), Wf1[D:], Wb1[D:])

    aggp1 = _sc_agg(tf1, tb1, mf, mb, i0, i1, None,
                    zrows, None, None, layer0=False)

    h2 = _tc_update1(aggp1, cntp, h1, row(lnng1), row(lnnb1))

    d2p = _sc_score(h2, rtp, i0, i1, ir)
    out = _tc_final(d2p, qf)
    return out[:E]


# speedup vs baseline: 1.8798x; 1.0984x over previous
import jax
import jax.numpy as jnp
from jax import lax
from jax.experimental import pallas as pl
from jax.experimental.pallas import tpu as pltpu
from jax.experimental.pallas import tpu_sc as plsc

D = 128
N = 10000
N_PAD = 10240
TRASH = 10000
E = 160000
E_PAD = 163840
REL_PAD = 512
NC = 2
NS = 16
NW = NC * NS
CHUNK = 128
CPT = E_PAD // (NW * CHUNK)
AGG_CHUNK = 64
AGG_CPT = E_PAD // (NW * AGG_CHUNK)
NB = 512

_F32 = jnp.float32


def _lrelu(v):
    return jnp.maximum(v, 0.0) + 0.01 * jnp.minimum(v, 0.0)


def _lnorm(v, g, b, eps=1e-5):
    mu = jnp.mean(v, axis=-1, keepdims=True)
    var = jnp.mean((v - mu) ** 2, axis=-1, keepdims=True)
    return (v - mu) / jnp.sqrt(var + eps) * g + b


def _full(shape):
    return pl.BlockSpec(shape, lambda i: tuple(0 for _ in shape))


def _tc_prep(xp, w_ent, b_ent2, wf0h, wb0h):
    def body(x_ref, we_ref, be_ref, wf_ref, wb_ref, h_ref, tf_ref, tb_ref):
        h = _lrelu(jnp.dot(x_ref[...], we_ref[...],
                           preferred_element_type=_F32) + be_ref[...])
        h_ref[...] = h
        tf_ref[...] = jnp.dot(h, wf_ref[...], preferred_element_type=_F32)
        tb_ref[...] = jnp.dot(h, wb_ref[...], preferred_element_type=_F32)

    out = jax.ShapeDtypeStruct((N_PAD, D), _F32)
    return pl.pallas_call(
        body,
        grid=(N_PAD // NB,),
        in_specs=[pl.BlockSpec((NB, D), lambda i: (i, 0)),
                  _full((D, D)), _full((1, D)), _full((D, D)), _full((D, D))],
        out_specs=[pl.BlockSpec((NB, D), lambda i: (i, 0))] * 3,
        out_shape=[out, out, out],
    )(xp, w_ent, b_ent2, wf0h, wb0h)


def _tc_rel(relp, wf0e, bf02, wb0e, bb02, we0m, be02):
    def body(r_ref, wf_ref, bf_ref, wb_ref, bb_ref, we_ref, be_ref,
             rf_ref, rb_ref, re_ref):
        r = r_ref[...]
        rf_ref[...] = jnp.dot(r, wf_ref[...], preferred_element_type=_F32) + bf_ref[...]
        rb_ref[...] = jnp.dot(r, wb_ref[...], preferred_element_type=_F32) + bb_ref[...]
        re_ref[...] = jnp.dot(r, we_ref[...], preferred_element_type=_F32) + be_ref[...]

    out = jax.ShapeDtypeStruct((REL_PAD, D), _F32)
    return pl.pallas_call(
        body,
        grid=(1,),
        in_specs=[_full((REL_PAD, D)), _full((D, D)), _full((1, D)),
                  _full((D, D)), _full((1, D)), _full((D, D)), _full((1, D))],
        out_specs=[_full((REL_PAD, D))] * 3,
        out_shape=[out, out, out],
    )(relp, wf0e, bf02, wb0e, bb02, we0m, be02)


def _tc_update0(aggp, cntp, h0, g2, b2, we0h, we0t, wf1h, bf12, wb1h, bb12):
    def body(a_ref, c_ref, h_ref, g_ref, b_ref, weh_ref, wet_ref,
             wf_ref, bf_ref, wb_ref, bb_ref,
             h1_ref, teh_ref, tet_ref, tf_ref, tb_ref):
        agg = a_ref[0] + a_ref[1]
        cnt = (c_ref[0] + c_ref[1]).reshape(NB, 1)
        h1 = _lnorm(_lrelu(agg / jnp.maximum(cnt, 1.0)) + h_ref[...],
                    g_ref[...], b_ref[...])
        h1_ref[...] = h1
        teh_ref[...] = jnp.dot(h1, weh_ref[...], preferred_element_type=_F32)
        tet_ref[...] = jnp.dot(h1, wet_ref[...], preferred_element_type=_F32)
        tf_ref[...] = jnp.dot(h1, wf_ref[...], preferred_element_type=_F32) + bf_ref[...]
        tb_ref[...] = jnp.dot(h1, wb_ref[...], preferred_element_type=_F32) + bb_ref[...]

    out = jax.ShapeDtypeStruct((N_PAD, D), _F32)
    return pl.pallas_call(
        body,
        grid=(N_PAD // NB,),
        in_specs=[pl.BlockSpec((NC, NB, D), lambda i: (0, i, 0)),
                  pl.BlockSpec((NC, NB), lambda i: (0, i)),
                  pl.BlockSpec((NB, D), lambda i: (i, 0)),
                  _full((1, D)), _full((1, D)), _full((D, D)), _full((D, D)),
                  _full((D, D)), _full((1, D)), _full((D, D)), _full((1, D))],
        out_specs=[pl.BlockSpec((NB, D), lambda i: (i, 0))] * 5,
        out_shape=[out] * 5,
    )(aggp, cntp, h0, g2, b2, we0h, we0t, wf1h, bf12, wb1h, bb12)


def _tc_update1(aggp, cntp, h1, g2, b2):
    def body(a_ref, c_ref, h_ref, g_ref, b_ref, h2_ref):
        agg = a_ref[0] + a_ref[1]
        cnt = (c_ref[0] + c_ref[1]).reshape(NB, 1)
        h2_ref[...] = _lnorm(_lrelu(agg / jnp.maximum(cnt, 1.0)) + h_ref[...],
                             g_ref[...], b_ref[...])

    return pl.pallas_call(
        body,
        grid=(N_PAD // NB,),
        in_specs=[pl.BlockSpec((NC, NB, D), lambda i: (0, i, 0)),
                  pl.BlockSpec((NC, NB), lambda i: (0, i)),
                  pl.BlockSpec((NB, D), lambda i: (i, 0)),
                  _full((1, D)), _full((1, D))],
        out_specs=pl.BlockSpec((NB, D), lambda i: (i, 0)),
        out_shape=jax.ShapeDtypeStruct((N_PAD, D), _F32),
    )(aggp, cntp, h1, g2, b2)


def _tc_msg1(qe, g2, b2, wf1e, wb1e):
    def body(q_ref, g_ref, b_ref, wf_ref, wb_ref, mf_ref, mb_ref):
        e1 = _lnorm(q_ref[...], g_ref[...], b_ref[...])
        mf_ref[...] = jnp.dot(e1, wf_ref[...], preferred_element_type=_F32)
        mb_ref[...] = jnp.dot(e1, wb_ref[...], preferred_element_type=_F32)

    out = jax.ShapeDtypeStruct((E_PAD, D), _F32)
    return pl.pallas_call(
        body,
        grid=(E_PAD // NB,),
        in_specs=[pl.BlockSpec((NB, D), lambda i: (i, 0)),
                  _full((1, D)), _full((1, D)), _full((D, D)), _full((D, D))],
        out_specs=[pl.BlockSpec((NB, D), lambda i: (i, 0))] * 2,
        out_shape=[out, out],
    )(qe, g2, b2, wf1e, wb1e)


def _tc_final(d2p, qf):
    def body(d_ref, q_ref, o_ref):
        s = jnp.sum(d_ref[...], axis=1)
        o_ref[...] = -jnp.sqrt(s + 1e-12) * q_ref[...]

    return pl.pallas_call(
        body,
        grid=(E_PAD // NB,),
        in_specs=[pl.BlockSpec((NB, 16), lambda i: (i, 0)),
                  pl.BlockSpec((NB,), lambda i: (i,))],
        out_specs=pl.BlockSpec((NB,), lambda i: (i,)),
        out_shape=jax.ShapeDtypeStruct((E_PAD,), _F32),
    )(d2p, qf)


def _sc_mesh():
    return plsc.VectorSubcoreMesh(core_axis_name="c", subcore_axis_name="s")


def _sc_agg(tf, tb, sf, sb, i0, i1, ir, zrows, zcnt, orows, layer0):
    outs = [jax.ShapeDtypeStruct((NC * N_PAD, D), _F32)]
    scratch = [
        pltpu.VMEM((CHUNK, D), _F32),
        pltpu.VMEM((CHUNK, D), _F32),
        pltpu.VMEM((CHUNK,), _F32),
        pltpu.VMEM((CHUNK,), jnp.int32),
        pltpu.VMEM((CHUNK,), jnp.int32),
        pltpu.VMEM((CHUNK,), jnp.int32),
        pltpu.VMEM_SHARED((N_PAD, D), _F32),
        pltpu.VMEM_SHARED((N_PAD,), _F32),
        pltpu.SemaphoreType.DMA,
        pltpu.SemaphoreType.DMA,
    ]
    if layer0:
        outs.append(jax.ShapeDtypeStruct((NC * N_PAD,), _F32))

    def body(*refs):
        if layer0:
            (tf_h, tb_h, sf_h, sb_h, i0_h, i1_h, ir_h, z_h, zc_h, o_h,
             agg_o, cnt_o,
             bufA, bufB, ones, ib0, ib1, ibr, agg_sh, cnt_sh, semA, semB) = refs
        else:
            (tf_h, tb_h, sf_h, sb_h, i0_h, i1_h, z_h, agg_o,
             bufA, bufB, ones, ib0, ib1, ibr, agg_sh, cnt_sh, semA, semB) = refs
            ir_h = zc_h = o_h = cnt_o = None
        cid = lax.axis_index("c")
        sid = lax.axis_index("s")
        wid = sid * NC + cid
        rows = N_PAD // NS

        pltpu.sync_copy(z_h.at[pl.ds(0, rows)], agg_sh.at[pl.ds(sid * rows, rows)])
        if layer0:
            pltpu.sync_copy(zc_h.at[pl.ds(0, rows)],
                            cnt_sh.at[pl.ds(sid * rows, rows)])
            pltpu.sync_copy(o_h.at[pl.ds(0, CHUNK)], ones)

        plsc.subcore_barrier()

        base0 = wid * CPT * CHUNK

        @pl.loop(0, CPT)
        def _(k):
            base = base0 + k * CHUNK
            pltpu.sync_copy(i0_h.at[pl.ds(base, CHUNK)], ib0)
            pltpu.sync_copy(i1_h.at[pl.ds(base, CHUNK)], ib1)
            if layer0:
                pltpu.sync_copy(ir_h.at[pl.ds(base, CHUNK)], ibr)
            cA = pltpu.async_copy(tf_h.at[ib0], bufA, semA)
            if layer0:
                cB = pltpu.async_copy(sf_h.at[ibr], bufB, semB)
            else:
                cB = pltpu.async_copy(sf_h.at[pl.ds(base, CHUNK)], bufB, semB)
            cA.wait()
            pltpu.sync_copy(bufA, agg_sh.at[ib1], add=True)
            cB.wait()
            cA = pltpu.async_copy(tb_h.at[ib1], bufA, semA)
            pltpu.sync_copy(bufB, agg_sh.at[ib1], add=True)
            if layer0:
                cB = pltpu.async_copy(sb_h.at[ibr], bufB, semB)
            else:
                cB = pltpu.async_copy(sb_h.at[pl.ds(base, CHUNK)], bufB, semB)
            cA.wait()
            pltpu.sync_copy(bufA, agg_sh.at[ib0], add=True)
            cB.wait()
            pltpu.sync_copy(bufB, agg_sh.at[ib0], add=True)
            if layer0:
                pltpu.sync_copy(ones, cnt_sh.at[ib1], add=True)
                pltpu.sync_copy(ones, cnt_sh.at[ib0], add=True)

        plsc.subcore_barrier()

        @pl.loop(0, rows // CHUNK)
        def _(j):
            lo = sid * rows + j * CHUNK
            pltpu.sync_copy(agg_sh.at[pl.ds(lo, CHUNK)],
                            agg_o.at[pl.ds(cid * N_PAD + lo, CHUNK)])
        if layer0:
            pltpu.sync_copy(cnt_sh.at[pl.ds(sid * rows, rows)],
                            cnt_o.at[pl.ds(cid * N_PAD + sid * rows, rows)])

    k = pl.kernel(body, out_type=tuple(outs) if layer0 else outs[0],
                  mesh=_sc_mesh(), scratch_types=scratch)
    if layer0:
        a, c = k(tf, tb, sf, sb, i0, i1, ir, zrows, zcnt, orows)
        return a.reshape(NC, N_PAD, D), c.reshape(NC, N_PAD)
    return k(tf, tb, sf, sb, i0, i1, zrows).reshape(NC, N_PAD, D)


def _sc_edgeq(teh, tet, re0, relp, i0, i1, ir):
    CH = 80
    CPQ = E_PAD // (NW * CH)

    def body(teh_h, tet_h, re_h, rel_h, i0_h, i1_h, ir_h, q_o, *scr):
        sets = (scr[0:11], scr[11:22])
        sid = lax.axis_index("s")
        wid = sid * NC + lax.axis_index("c")
        base0 = wid * CPQ * CH

        def fire(k, st):
            bA, bB, bC, bD, ia, ib_, ic, sa, sb, sc_, sd_ = st
            base = base0 + k * CH
            pltpu.sync_copy(i0_h.at[pl.ds(base, CH)], ia)
            pltpu.sync_copy(i1_h.at[pl.ds(base, CH)], ib_)
            pltpu.sync_copy(ir_h.at[pl.ds(base, CH)], ic)
            pltpu.async_copy(teh_h.at[ia], bA, sa)
            pltpu.async_copy(tet_h.at[ib_], bB, sb)
            pltpu.async_copy(re_h.at[ic], bC, sc_)
            pltpu.async_copy(rel_h.at[ic], bD, sd_)

        def wait(st):
            bA, bB, bC, bD, ia, ib_, ic, sa, sb, sc_, sd_ = st
            pltpu.make_async_copy(teh_h.at[ia], bA, sa).wait()
            pltpu.make_async_copy(tet_h.at[ib_], bB, sb).wait()
            pltpu.make_async_copy(re_h.at[ic], bC, sc_).wait()
            pltpu.make_async_copy(rel_h.at[ic], bD, sd_).wait()

        def process(k, st):
            bA, bB, bC, bD = st[0], st[1], st[2], st[3]

            @pl.loop(0, CH)
            def _(e):
                for c in range(D // 16):
                    sl = pl.ds(c * 16, 16)
                    g = bA[e, sl] + bB[e, sl] + bC[e, sl]
                    bA[e, sl] = (jnp.maximum(g, 0.0)
                                 + 0.01 * jnp.minimum(g, 0.0)) + bD[e, sl]

            pltpu.sync_copy(bA, q_o.at[pl.ds(base0 + k * CH, CH)])

        fire(0, sets[0])

        @pl.loop(0, CPQ // 2)
        def _(kk):
            k = kk * 2
            fire(k + 1, sets[1])
            wait(sets[0])
            process(k, sets[0])

            @pl.when(k + 2 < CPQ)
            def _():
                fire(k + 2, sets[0])

            wait(sets[1])
            process(k + 1, sets[1])

    one_set = ([pltpu.VMEM((CH, D), _F32)] * 4
               + [pltpu.VMEM((CH,), jnp.int32)] * 3
               + [pltpu.SemaphoreType.DMA] * 4)
    k = pl.kernel(body, out_type=jax.ShapeDtypeStruct((E_PAD, D), _F32),
                  mesh=_sc_mesh(), scratch_types=one_set + one_set)
    return k(teh, tet, re0, relp, i0, i1, ir)


def _sc_score(h2, rtp, i0, i1, ir):
    CH = 80
    CPQ = E_PAD // (NW * CH)

    def body(h_h, rt_h, i0_h, i1_h, ir_h, d2_o, *scr):
        sets = (scr[0:10], scr[10:20])
        sid = lax.axis_index("s")
        wid = sid * NC + lax.axis_index("c")
        base0 = wid * CPQ * CH

        def fire(k, st):
            bA, bB, bC, sb_, ia, ib_, ic, sa, sbm, sc_ = st
            base = base0 + k * CH
            pltpu.sync_copy(i0_h.at[pl.ds(base, CH)], ia)
            pltpu.sync_copy(i1_h.at[pl.ds(base, CH)], ib_)
            pltpu.sync_copy(ir_h.at[pl.ds(base, CH)], ic)
            pltpu.async_copy(h_h.at[ia], bA, sa)
            pltpu.async_copy(h_h.at[ib_], bB, sbm)
            pltpu.async_copy(rt_h.at[ic], bC, sc_)

        def wait(st):
            bA, bB, bC, sb_, ia, ib_, ic, sa, sbm, sc_ = st
            pltpu.make_async_copy(h_h.at[ia], bA, sa).wait()
            pltpu.make_async_copy(h_h.at[ib_], bB, sbm).wait()
            pltpu.make_async_copy(rt_h.at[ic], bC, sc_).wait()

        def process(k, st):
            bA, bB, bC, sb_ = st[0], st[1], st[2], st[3]

            @pl.loop(0, CH)
            def _(e):
                acc = jnp.zeros((16,), _F32)
                for c in range(D // 16):
                    sl = pl.ds(c * 16, 16)
                    dd = bA[e, sl] + bC[e, sl] - bB[e, sl]
                    acc = acc + dd * dd
                sb_[pl.ds(e * 16, 16)] = acc

            pltpu.sync_copy(sb_, d2_o.at[pl.ds((base0 + k * CH) * 16, CH * 16)])

        fire(0, sets[0])

        @pl.loop(0, CPQ // 2)
        def _(kk):
            k = kk * 2
            fire(k + 1, sets[1])
            wait(sets[0])
            process(k, sets[0])

            @pl.when(k + 2 < CPQ)
            def _():
                fire(k + 2, sets[0])

            wait(sets[1])
            process(k + 1, sets[1])

    one_set = ([pltpu.VMEM((CH, D), _F32)] * 3
               + [pltpu.VMEM((CH * 16,), _F32)]
               + [pltpu.VMEM((CH,), jnp.int32)] * 3
               + [pltpu.SemaphoreType.DMA] * 3)
    k = pl.kernel(body, out_type=jax.ShapeDtypeStruct((E_PAD * 16,), _F32),
                  mesh=_sc_mesh(), scratch_types=one_set + one_set)
    return k(h2, rtp, i0, i1, ir).reshape(E_PAD, 16)


def kernel(x, ht, r_tensor, queries, W_ent, b_ent, rel_emb, rel_transe,
           Wf0, bf0, Wb0, bb0, lnng0, lnnb0, We0, be0, lneg0, lneb0,
           Wf1, bf1, Wb1, bb1, lnng1, lnnb1, We1, be1, lneg1, lneb1):
    pad_e = E_PAD - E
    xp = jnp.pad(x, ((0, N_PAD - N), (0, 0)))
    i0 = jnp.concatenate([ht[:, 0], jnp.full((pad_e,), TRASH, jnp.int32)])
    i1 = jnp.concatenate([ht[:, 1], jnp.full((pad_e,), TRASH, jnp.int32)])
    ir = jnp.concatenate([r_tensor, jnp.zeros((pad_e,), jnp.int32)])
    relp = jnp.pad(rel_emb, ((0, REL_PAD - rel_emb.shape[0]), (0, 0)))
    rtp = jnp.pad(rel_transe, ((0, REL_PAD - rel_transe.shape[0]), (0, 0)))
    qf = jnp.pad(queries.astype(_F32), (0, pad_e))

    row = lambda v: v.reshape(1, D)

    h0, tf0, tb0 = _tc_prep(xp, W_ent, row(b_ent), Wf0[:D], Wb0[:D])
    rf0, rb0, re0 = _tc_rel(relp, Wf0[D:], row(bf0), Wb0[D:], row(bb0),
                            We0[D:2 * D], row(be0))

    zrows = jnp.zeros((N_PAD // NS, D), _F32)
    zcnt = jnp.zeros((N_PAD // NS,), _F32)
    orows = jnp.ones((CHUNK,), _F32)

    aggp, cntp = _sc_agg(tf0, tb0, rf0, rb0, i0, i1, ir,
                         zrows, zcnt, orows, layer0=True)

    h1, teh, tet, tf1, tb1 = _tc_update0(
        aggp, cntp, h0, row(lnng0), row(lnnb0),
        We0[:D], We0[2 * D:], Wf1[:D], row(bf1), Wb1[:D], row(bb1))

    qe = _sc_edgeq(teh, tet, re0, relp, i0, i1, ir)
    mf, mb = _tc_msg1(qe, row(lneg0), row(lneb0), Wf1[D:], Wb1[D:])

    aggp1 = _sc_agg(tf1, tb1, mf, mb, i0, i1, None,
                    zrows, None, None, layer0=False)

    h2 = _tc_update1(aggp1, cntp, h1, row(lnng1), row(lnnb1))

    d2p = _sc_score(h2, rtp, i0, i1, ir)
    out = _tc_final(d2p, qf)
    return out[:E]


# speedup vs baseline: 2.1847x; 1.1622x over previous
import jax
import jax.numpy as jnp
from jax import lax
from jax.experimental import pallas as pl
from jax.experimental.pallas import tpu as pltpu
from jax.experimental.pallas import tpu_sc as plsc

D = 128
N = 10000
N_PAD = 10240
TRASH = 10000
E = 160000
E_PAD = 163840
REL_PAD = 512
NC = 2
NS = 16
NW = NC * NS
CHUNK = 128
CPT = E_PAD // (NW * CHUNK)
AGG_CHUNK = 64
AGG_CPT = E_PAD // (NW * AGG_CHUNK)
NB = 512

_F32 = jnp.float32


def _lrelu(v):
    return jnp.maximum(v, 0.0) + 0.01 * jnp.minimum(v, 0.0)


def _lnorm(v, g, b, eps=1e-5):
    mu = jnp.mean(v, axis=-1, keepdims=True)
    var = jnp.mean((v - mu) ** 2, axis=-1, keepdims=True)
    return (v - mu) / jnp.sqrt(var + eps) * g + b


def _full(shape):
    return pl.BlockSpec(shape, lambda i: tuple(0 for _ in shape))


def _tc_prep(xp, w_ent, b_ent2, wf0h, wb0h):
    def body(x_ref, we_ref, be_ref, wf_ref, wb_ref, h_ref, tf_ref, tb_ref):
        h = _lrelu(jnp.dot(x_ref[...], we_ref[...],
                           preferred_element_type=_F32) + be_ref[...])
        h_ref[...] = h
        tf_ref[...] = jnp.dot(h, wf_ref[...], preferred_element_type=_F32)
        tb_ref[...] = jnp.dot(h, wb_ref[...], preferred_element_type=_F32)

    out = jax.ShapeDtypeStruct((N_PAD, D), _F32)
    return pl.pallas_call(
        body,
        grid=(N_PAD // NB,),
        in_specs=[pl.BlockSpec((NB, D), lambda i: (i, 0)),
                  _full((D, D)), _full((1, D)), _full((D, D)), _full((D, D))],
        out_specs=[pl.BlockSpec((NB, D), lambda i: (i, 0))] * 3,
        out_shape=[out, out, out],
    )(xp, w_ent, b_ent2, wf0h, wb0h)


def _tc_rel(relp, wf0e, bf02, wb0e, bb02, we0m, be02):
    def body(r_ref, wf_ref, bf_ref, wb_ref, bb_ref, we_ref, be_ref,
             rf_ref, rb_ref, re_ref):
        r = r_ref[...]
        rf_ref[...] = jnp.dot(r, wf_ref[...], preferred_element_type=_F32) + bf_ref[...]
        rb_ref[...] = jnp.dot(r, wb_ref[...], preferred_element_type=_F32) + bb_ref[...]
        re_ref[...] = jnp.dot(r, we_ref[...], preferred_element_type=_F32) + be_ref[...]

    out = jax.ShapeDtypeStruct((REL_PAD, D), _F32)
    return pl.pallas_call(
        body,
        grid=(1,),
        in_specs=[_full((REL_PAD, D)), _full((D, D)), _full((1, D)),
                  _full((D, D)), _full((1, D)), _full((D, D)), _full((1, D))],
        out_specs=[_full((REL_PAD, D))] * 3,
        out_shape=[out, out, out],
    )(relp, wf0e, bf02, wb0e, bb02, we0m, be02)


def _tc_update0(aggp, cntp, h0, g2, b2, we0h, we0t, wf1h, bf12, wb1h, bb12):
    def body(a_ref, c_ref, h_ref, g_ref, b_ref, weh_ref, wet_ref,
             wf_ref, bf_ref, wb_ref, bb_ref,
             h1_ref, teh_ref, tet_ref, tf_ref, tb_ref):
        agg = a_ref[0] + a_ref[1]
        cnt = (c_ref[0] + c_ref[1]).reshape(NB, 1)
        h1 = _lnorm(_lrelu(agg / jnp.maximum(cnt, 1.0)) + h_ref[...],
                    g_ref[...], b_ref[...])
        h1_ref[...] = h1
        teh_ref[...] = jnp.dot(h1, weh_ref[...], preferred_element_type=_F32)
        tet_ref[...] = jnp.dot(h1, wet_ref[...], preferred_element_type=_F32)
        tf_ref[...] = jnp.dot(h1, wf_ref[...], preferred_element_type=_F32) + bf_ref[...]
        tb_ref[...] = jnp.dot(h1, wb_ref[...], preferred_element_type=_F32) + bb_ref[...]

    out = jax.ShapeDtypeStruct((N_PAD, D), _F32)
    return pl.pallas_call(
        body,
        grid=(N_PAD // NB,),
        in_specs=[pl.BlockSpec((NC, NB, D), lambda i: (0, i, 0)),
                  pl.BlockSpec((NC, NB), lambda i: (0, i)),
                  pl.BlockSpec((NB, D), lambda i: (i, 0)),
                  _full((1, D)), _full((1, D)), _full((D, D)), _full((D, D)),
                  _full((D, D)), _full((1, D)), _full((D, D)), _full((1, D))],
        out_specs=[pl.BlockSpec((NB, D), lambda i: (i, 0))] * 5,
        out_shape=[out] * 5,
    )(aggp, cntp, h0, g2, b2, we0h, we0t, wf1h, bf12, wb1h, bb12)


def _tc_update1(aggp, cntp, h1, g2, b2):
    def body(a_ref, c_ref, h_ref, g_ref, b_ref, h2_ref):
        agg = a_ref[0] + a_ref[1]
        cnt = (c_ref[0] + c_ref[1]).reshape(NB, 1)
        h2_ref[...] = _lnorm(_lrelu(agg / jnp.maximum(cnt, 1.0)) + h_ref[...],
                             g_ref[...], b_ref[...])

    return pl.pallas_call(
        body,
        grid=(N_PAD // NB,),
        in_specs=[pl.BlockSpec((NC, NB, D), lambda i: (0, i, 0)),
                  pl.BlockSpec((NC, NB), lambda i: (0, i)),
                  pl.BlockSpec((NB, D), lambda i: (i, 0)),
                  _full((1, D)), _full((1, D))],
        out_specs=pl.BlockSpec((NB, D), lambda i: (i, 0)),
        out_shape=jax.ShapeDtypeStruct((N_PAD, D), _F32),
    )(aggp, cntp, h1, g2, b2)


def _tc_msg1(qe, g2, b2, wf1e, wb1e):
    def body(q_ref, g_ref, b_ref, wf_ref, wb_ref, mf_ref, mb_ref):
        e1 = _lnorm(q_ref[...], g_ref[...], b_ref[...])
        mf_ref[...] = jnp.dot(e1, wf_ref[...], preferred_element_type=_F32)
        mb_ref[...] = jnp.dot(e1, wb_ref[...], preferred_element_type=_F32)

    out = jax.ShapeDtypeStruct((E_PAD, D), _F32)
    return pl.pallas_call(
        body,
        grid=(E_PAD // NB,),
        in_specs=[pl.BlockSpec((NB, D), lambda i: (i, 0)),
                  _full((1, D)), _full((1, D)), _full((D, D)), _full((D, D))],
        out_specs=[pl.BlockSpec((NB, D), lambda i: (i, 0))] * 2,
        out_shape=[out, out],
    )(qe, g2, b2, wf1e, wb1e)


def _tc_final(d2p, qf):
    def body(d_ref, q_ref, o_ref):
        s = jnp.sum(d_ref[...], axis=1)
        o_ref[...] = -jnp.sqrt(s + 1e-12) * q_ref[...]

    return pl.pallas_call(
        body,
        grid=(E_PAD // NB,),
        in_specs=[pl.BlockSpec((NB, 16), lambda i: (i, 0)),
                  pl.BlockSpec((NB,), lambda i: (i,))],
        out_specs=pl.BlockSpec((NB,), lambda i: (i,)),
        out_shape=jax.ShapeDtypeStruct((E_PAD,), _F32),
    )(d2p, qf)


def _sc_mesh():
    return plsc.VectorSubcoreMesh(core_axis_name="c", subcore_axis_name="s")


def _sc_agg(tf, tb, sf, sb, i0, i1, ir, zrows, zcnt, orows, layer0):
    CH = 64
    CPA = E_PAD // (NW * CH)
    outs = [jax.ShapeDtypeStruct((NC * N_PAD, D), _F32)]
    one_set = ([pltpu.VMEM((CH, D), _F32)] * 2
               + [pltpu.VMEM((CH,), jnp.int32)] * 3
               + [pltpu.SemaphoreType.DMA] * 2)
    scratch = (one_set + one_set
               + [pltpu.VMEM((CH,), _F32),
                  pltpu.VMEM_SHARED((N_PAD, D), _F32),
                  pltpu.VMEM_SHARED((N_PAD,), _F32)])
    if layer0:
        outs.append(jax.ShapeDtypeStruct((NC * N_PAD,), _F32))

    def body(*refs):
        if layer0:
            (tf_h, tb_h, sf_h, sb_h, i0_h, i1_h, ir_h, z_h, zc_h, o_h,
             agg_o, cnt_o) = refs[:12]
            scr = refs[12:]
        else:
            (tf_h, tb_h, sf_h, sb_h, i0_h, i1_h, z_h, agg_o) = refs[:8]
            scr = refs[8:]
            ir_h = zc_h = o_h = cnt_o = None
        sets = (scr[0:7], scr[7:14])
        ones, agg_sh, cnt_sh = scr[14], scr[15], scr[16]
        cid = lax.axis_index("c")
        sid = lax.axis_index("s")
        wid = sid * NC + cid
        rows = N_PAD // NS

        pltpu.sync_copy(z_h.at[pl.ds(0, rows)],
                        agg_sh.at[pl.ds(sid * rows, rows)])
        if layer0:
            pltpu.sync_copy(zc_h.at[pl.ds(0, rows)],
                            cnt_sh.at[pl.ds(sid * rows, rows)])
            pltpu.sync_copy(o_h.at[pl.ds(0, CH)], ones)

        plsc.subcore_barrier()

        base0 = wid * CPA * CH

        def fire(m, st, fwd):
            bA, bB, ia0, ia1, iar, sa, sb_ = st
            base = base0 + m * CH
            pltpu.sync_copy(i0_h.at[pl.ds(base, CH)], ia0)
            pltpu.sync_copy(i1_h.at[pl.ds(base, CH)], ia1)
            node_h, edge_h = (tf_h, sf_h) if fwd else (tb_h, sb_h)
            pltpu.async_copy(node_h.at[ia0 if fwd else ia1], bA, sa)
            if layer0:
                pltpu.sync_copy(ir_h.at[pl.ds(base, CH)], iar)
                pltpu.async_copy(edge_h.at[iar], bB, sb_)
            else:
                pltpu.async_copy(edge_h.at[pl.ds(base, CH)], bB, sb_)

        def process(st, fwd):
            bA, bB, ia0, ia1, iar, sa, sb_ = st
            src = ia0 if fwd else ia1
            dst = ia1 if fwd else ia0
            node_h = tf_h if fwd else tb_h
            pltpu.make_async_copy(node_h.at[src], bA, sa).wait()
            pltpu.make_async_copy(node_h.at[src], bB, sb_).wait()
            pltpu.sync_copy(bA, agg_sh.at[dst], add=True)
            pltpu.sync_copy(bB, agg_sh.at[dst], add=True)
            if layer0:
                pltpu.sync_copy(ones, cnt_sh.at[dst], add=True)

        fire(0, sets[0], True)

        @pl.loop(0, CPA)
        def _(m):
            fire(m, sets[1], False)
            process(sets[0], True)

            @pl.when(m + 1 < CPA)
            def _():
                fire(m + 1, sets[0], True)

            process(sets[1], False)

        plsc.subcore_barrier()

        @pl.loop(0, rows // CHUNK)
        def _(j):
            lo = sid * rows + j * CHUNK
            pltpu.sync_copy(agg_sh.at[pl.ds(lo, CHUNK)],
                            agg_o.at[pl.ds(cid * N_PAD + lo, CHUNK)])
        if layer0:
            pltpu.sync_copy(cnt_sh.at[pl.ds(sid * rows, rows)],
                            cnt_o.at[pl.ds(cid * N_PAD + sid * rows, rows)])

    k = pl.kernel(body, out_type=tuple(outs) if layer0 else outs[0],
                  mesh=_sc_mesh(), scratch_types=scratch)
    if layer0:
        a, c = k(tf, tb, sf, sb, i0, i1, ir, zrows, zcnt, orows)
        return a.reshape(NC, N_PAD, D), c.reshape(NC, N_PAD)
    return k(tf, tb, sf, sb, i0, i1, zrows).reshape(NC, N_PAD, D)


def _sc_edgeq(teh, tet, re0, relp, i0, i1, ir):
    CH = 80
    CPQ = E_PAD // (NW * CH)

    def body(teh_h, tet_h, re_h, rel_h, i0_h, i1_h, ir_h, q_o, *scr):
        sets = (scr[0:11], scr[11:22])
        sid = lax.axis_index("s")
        wid = sid * NC + lax.axis_index("c")
        base0 = wid * CPQ * CH

        def fire(k, st):
            bA, bB, bC, bD, ia, ib_, ic, sa, sb, sc_, sd_ = st
            base = base0 + k * CH
            pltpu.sync_copy(i0_h.at[pl.ds(base, CH)], ia)
            pltpu.sync_copy(i1_h.at[pl.ds(base, CH)], ib_)
            pltpu.sync_copy(ir_h.at[pl.ds(base, CH)], ic)
            pltpu.async_copy(teh_h.at[ia], bA, sa)
            pltpu.async_copy(tet_h.at[ib_], bB, sb)
            pltpu.async_copy(re_h.at[ic], bC, sc_)
            pltpu.async_copy(rel_h.at[ic], bD, sd_)

        def wait(st):
            bA, bB, bC, bD, ia, ib_, ic, sa, sb, sc_, sd_ = st
            pltpu.make_async_copy(teh_h.at[ia], bA, sa).wait()
            pltpu.make_async_copy(tet_h.at[ib_], bB, sb).wait()
            pltpu.make_async_copy(re_h.at[ic], bC, sc_).wait()
            pltpu.make_async_copy(rel_h.at[ic], bD, sd_).wait()

        def process(k, st):
            bA, bB, bC, bD = st[0], st[1], st[2], st[3]

            @pl.loop(0, CH)
            def _(e):
                for c in range(D // 16):
                    sl = pl.ds(c * 16, 16)
                    g = bA[e, sl] + bB[e, sl] + bC[e, sl]
                    bA[e, sl] = (jnp.maximum(g, 0.0)
                                 + 0.01 * jnp.minimum(g, 0.0)) + bD[e, sl]

            pltpu.sync_copy(bA, q_o.at[pl.ds(base0 + k * CH, CH)])

        fire(0, sets[0])

        @pl.loop(0, CPQ // 2)
        def _(kk):
            k = kk * 2
            fire(k + 1, sets[1])
            wait(sets[0])
            process(k, sets[0])

            @pl.when(k + 2 < CPQ)
            def _():
                fire(k + 2, sets[0])

            wait(sets[1])
            process(k + 1, sets[1])

    one_set = ([pltpu.VMEM((CH, D), _F32)] * 4
               + [pltpu.VMEM((CH,), jnp.int32)] * 3
               + [pltpu.SemaphoreType.DMA] * 4)
    k = pl.kernel(body, out_type=jax.ShapeDtypeStruct((E_PAD, D), _F32),
                  mesh=_sc_mesh(), scratch_types=one_set + one_set)
    return k(teh, tet, re0, relp, i0, i1, ir)


def _sc_score(h2, rtp, i0, i1, ir):
    CH = 80
    CPQ = E_PAD // (NW * CH)

    def body(h_h, rt_h, i0_h, i1_h, ir_h, d2_o, *scr):
        sets = (scr[0:10], scr[10:20])
        sid = lax.axis_index("s")
        wid = sid * NC + lax.axis_index("c")
        base0 = wid * CPQ * CH

        def fire(k, st):
            bA, bB, bC, sb_, ia, ib_, ic, sa, sbm, sc_ = st
            base = base0 + k * CH
            pltpu.sync_copy(i0_h.at[pl.ds(base, CH)], ia)
            pltpu.sync_copy(i1_h.at[pl.ds(base, CH)], ib_)
            pltpu.sync_copy(ir_h.at[pl.ds(base, CH)], ic)
            pltpu.async_copy(h_h.at[ia], bA, sa)
            pltpu.async_copy(h_h.at[ib_], bB, sbm)
            pltpu.async_copy(rt_h.at[ic], bC, sc_)

        def wait(st):
            bA, bB, bC, sb_, ia, ib_, ic, sa, sbm, sc_ = st
            pltpu.make_async_copy(h_h.at[ia], bA, sa).wait()
            pltpu.make_async_copy(h_h.at[ib_], bB, sbm).wait()
            pltpu.make_async_copy(rt_h.at[ic], bC, sc_).wait()

        def process(k, st):
            bA, bB, bC, sb_ = st[0], st[1], st[2], st[3]

            @pl.loop(0, CH)
            def _(e):
                acc = jnp.zeros((16,), _F32)
                for c in range(D // 16):
                    sl = pl.ds(c * 16, 16)
                    dd = bA[e, sl] + bC[e, sl] - bB[e, sl]
                    acc = acc + dd * dd
                sb_[pl.ds(e * 16, 16)] = acc

            pltpu.sync_copy(sb_, d2_o.at[pl.ds((base0 + k * CH) * 16, CH * 16)])

        fire(0, sets[0])

        @pl.loop(0, CPQ // 2)
        def _(kk):
            k = kk * 2
            fire(k + 1, sets[1])
            wait(sets[0])
            process(k, sets[0])

            @pl.when(k + 2 < CPQ)
            def _():
                fire(k + 2, sets[0])

            wait(sets[1])
            process(k + 1, sets[1])

    one_set = ([pltpu.VMEM((CH, D), _F32)] * 3
               + [pltpu.VMEM((CH * 16,), _F32)]
               + [pltpu.VMEM((CH,), jnp.int32)] * 3
               + [pltpu.SemaphoreType.DMA] * 3)
    k = pl.kernel(body, out_type=jax.ShapeDtypeStruct((E_PAD * 16,), _F32),
                  mesh=_sc_mesh(), scratch_types=one_set + one_set)
    return k(h2, rtp, i0, i1, ir).reshape(E_PAD, 16)


def kernel(x, ht, r_tensor, queries, W_ent, b_ent, rel_emb, rel_transe,
           Wf0, bf0, Wb0, bb0, lnng0, lnnb0, We0, be0, lneg0, lneb0,
           Wf1, bf1, Wb1, bb1, lnng1, lnnb1, We1, be1, lneg1, lneb1):
    pad_e = E_PAD - E
    xp = jnp.pad(x, ((0, N_PAD - N), (0, 0)))
    i0 = jnp.concatenate([ht[:, 0], jnp.full((pad_e,), TRASH, jnp.int32)])
    i1 = jnp.concatenate([ht[:, 1], jnp.full((pad_e,), TRASH, jnp.int32)])
    ir = jnp.concatenate([r_tensor, jnp.zeros((pad_e,), jnp.int32)])
    relp = jnp.pad(rel_emb, ((0, REL_PAD - rel_emb.shape[0]), (0, 0)))
    rtp = jnp.pad(rel_transe, ((0, REL_PAD - rel_transe.shape[0]), (0, 0)))
    qf = jnp.pad(queries.astype(_F32), (0, pad_e))

    row = lambda v: v.reshape(1, D)

    h0, tf0, tb0 = _tc_prep(xp, W_ent, row(b_ent), Wf0[:D], Wb0[:D])
    rf0, rb0, re0 = _tc_rel(relp, Wf0[D:], row(bf0), Wb0[D:], row(bb0),
                            We0[D:2 * D], row(be0))

    zrows = jnp.zeros((N_PAD // NS, D), _F32)
    zcnt = jnp.zeros((N_PAD // NS,), _F32)
    orows = jnp.ones((64,), _F32)

    aggp, cntp = _sc_agg(tf0, tb0, rf0, rb0, i0, i1, ir,
                         zrows, zcnt, orows, layer0=True)

    h1, teh, tet, tf1, tb1 = _tc_update0(
        aggp, cntp, h0, row(lnng0), row(lnnb0),
        We0[:D], We0[2 * D:], Wf1[:D], row(bf1), Wb1[:D], row(bb1))

    qe = _sc_edgeq(teh, tet, re0, relp, i0, i1, ir)
    mf, mb = _tc_msg1(qe, row(lneg0), row(lneb0), Wf1[D:], Wb1[D:])

    aggp1 = _sc_agg(tf1, tb1, mf, mb, i0, i1, None,
                    zrows, None, None, layer0=False)

    h2 = _tc_update1(aggp1, cntp, h1, row(lnng1), row(lnnb1))

    d2p = _sc_score(h2, rtp, i0, i1, ir)
    out = _tc_final(d2p, qf)
    return out[:E]
